# baseline (speedup 1.0000x reference)
import jax
import jax.numpy as jnp
from jax.experimental import pallas as pl
from jax.experimental.pallas import tpu as pltpu

_EPS = 1e-5
_BM = 320


def _norm_rows(z, f):
    zr = jnp.maximum(z, 0.0)
    mean = jnp.sum(zr, axis=1, keepdims=True) * (1.0 / f)
    diff = zr - mean
    var = jnp.sum(diff * diff, axis=1, keepdims=True) * (1.0 / max(f - 1, 1))
    return diff * pl.reciprocal(jnp.sqrt(var) + _EPS, approx=True)


def _fused(x_bf, a_hat, w0, b0, w1, b1, w2, b2, fo, num_cores):
    n, f_in = x_bf.shape
    fh = w0.shape[1]
    half = n // num_cores
    nb = half // _BM

    mesh = pltpu.create_tensorcore_mesh("core", num_cores=num_cores)

    out_init = jnp.zeros((n, fo), jnp.float32)
    pen_init = jnp.zeros((n, fh), jnp.float32)
    h1x_init = jnp.zeros((n, fh), jnp.bfloat16)

    def run(refs):
        (a_ref, x_ref, w0_ref, b0_ref, w1_ref, b1_ref, w2_ref, b2_ref,
         out_ref, pen_ref, h1x_ref) = refs

        @pl.core_map(
            mesh,
            compiler_params=pltpu.CompilerParams(
                vmem_limit_bytes=50 * 1024 * 1024),
            scratch_shapes=[
                pltpu.VMEM((half, n), jnp.bfloat16),
                pltpu.VMEM((n, fh), jnp.bfloat16),
                pltpu.VMEM((n, fh), jnp.bfloat16),
                pltpu.VMEM((n, fh), jnp.float32),
                pltpu.VMEM((2, _BM, n), jnp.float32),
                pltpu.VMEM((n, f_in), jnp.bfloat16),
                pltpu.VMEM((fh, fh), jnp.bfloat16),
                pltpu.VMEM((1, fh), jnp.float32),
                pltpu.VMEM((half, fh), jnp.float32),
                pltpu.VMEM((half, fo), jnp.float32),
                pltpu.SemaphoreType.REGULAR,
                pltpu.SemaphoreType.DMA((3,)),
                pltpu.SemaphoreType.DMA((2,)),
            ],
        )
        def _(abf, hfull, xw, xwf, a32, xv, wv, bv, penv, outv,
              bar_sem, sem_s, sem_a):
            core = jax.lax.axis_index("core")
            row0 = pl.multiple_of(core * half, _BM)

            def a_copy(i, slot):
                return pltpu.make_async_copy(
                    a_ref.at[pl.ds(row0 + i * _BM, _BM)], a32.at[slot],
                    sem_a.at[slot])

            cp_x = pltpu.make_async_copy(x_ref, xv, sem_s.at[0])
            cp_w = pltpu.make_async_copy(w0_ref, wv.at[:f_in], sem_s.at[1])
            cp_b = pltpu.make_async_copy(b0_ref, bv, sem_s.at[2])
            cp_x.start()
            cp_w.start()
            cp_b.start()
            a_copy(0, 0).start()
            a_copy(1, 1).start()
            cp_x.wait()
            cp_w.wait()
            cp_b.wait()

            xwf[...] = jnp.dot(xv[...], wv[:f_in],
                               preferred_element_type=jnp.float32)

            for i in range(nb):
                a_copy(i, i % 2).wait()
                sl = slice(i * _BM, (i + 1) * _BM)
                z = jnp.dot(a32[i % 2], xwf[...],
                            preferred_element_type=jnp.float32) + bv[...]
                abf[sl] = a32[i % 2].astype(jnp.bfloat16)
                if i + 2 < nb:
                    a_copy(i + 2, i % 2).start()
                hfull[pl.ds(row0 + i * _BM, _BM)] = _norm_rows(z, fh).astype(
                    jnp.bfloat16)
            cp_h = pltpu.make_async_copy(
                hfull.at[pl.ds(row0, half)],
                h1x_ref.at[pl.ds(row0, half)], sem_s.at[0])
            cp_h.start()
            cp_h.wait()
            pltpu.core_barrier(bar_sem, core_axis_name="core")
            for d in range(1, num_cores):
                ostart = jax.lax.rem(core + d, num_cores) * half
                cp = pltpu.make_async_copy(
                    h1x_ref.at[pl.ds(ostart, half)],
                    hfull.at[pl.ds(ostart, half)], sem_s.at[0])
                cp.start()
                cp.wait()

            cp_w = pltpu.make_async_copy(w1_ref, wv, sem_s.at[1])
            cp_b = pltpu.make_async_copy(b1_ref, bv, sem_s.at[2])
            cp_w.start()
            cp_b.start()
            cp_w.wait()
            cp_b.wait()
            xw[...] = jnp.dot(
                hfull[...], wv[...],
                preferred_element_type=jnp.float32).astype(jnp.bfloat16)
            z = jnp.dot(abf[...], xw[...],
                        preferred_element_type=jnp.float32) + bv[...]
            h2 = _norm_rows(z, fh)
            penv[...] = h2
            hfull[pl.ds(row0, half)] = h2.astype(jnp.bfloat16)
            cp_p = pltpu.make_async_copy(
                penv, pen_ref.at[pl.ds(row0, half)], sem_s.at[0])
            cp_p.start()
            cp_p.wait()
            pltpu.core_barrier(bar_sem, core_axis_name="core")
            for d in range(1, num_cores):
                ostart = jax.lax.rem(core + d, num_cores) * half
                cp = pltpu.make_async_copy(
                    pen_ref.at[pl.ds(ostart, half)], penv, sem_s.at[0])
                cp.start()
                cp.wait()
                hfull[pl.ds(ostart, half)] = penv[...].astype(jnp.bfloat16)

            cp_w = pltpu.make_async_copy(w2_ref, wv, sem_s.at[1])
            cp_b = pltpu.make_async_copy(b2_ref, bv, sem_s.at[2])
            cp_w.start()
            cp_b.start()
            cp_w.wait()
            cp_b.wait()
            xw[...] = jnp.dot(
                hfull[...], wv[...],
                preferred_element_type=jnp.float32).astype(jnp.bfloat16)
            z = jnp.dot(abf[...], xw[...],
                        preferred_element_type=jnp.float32) + bv[...]
            outv[...] = z[:, :fo]
            cp_o = pltpu.make_async_copy(
                outv, out_ref.at[pl.ds(row0, half)], sem_s.at[0])
            cp_o.start()
            cp_o.wait()

    states = pl.run_state(run)(
        (a_hat, x_bf, w0, b0, w1, b1, w2, b2, out_init, pen_init, h1x_init))
    return states[8], states[9]


def kernel(x, a_hat, W0, b0, W1, b1, W2, b2):
    n = x.shape[0]
    fh = W0.shape[1]
    fo = W2.shape[1]

    num_cores = getattr(jax.devices()[0], "num_cores", 1) or 1
    if n % (num_cores * _BM) != 0:
        num_cores = 1

    x_bf = x.astype(jnp.bfloat16)
    w0 = W0.astype(jnp.bfloat16)
    w1 = W1.astype(jnp.bfloat16)
    w2 = jnp.zeros((fh, fh), jnp.float32).at[:, :fo].set(W2).astype(
        jnp.bfloat16)
    b2p = jnp.zeros((1, fh), jnp.float32).at[:, :fo].set(
        b2.reshape(1, -1))

    out, pen = _fused(x_bf, a_hat, w0, b0.reshape(1, -1), w1,
                      b1.reshape(1, -1), w2, b2p, fo, num_cores)
    return out, pen


# speedup vs baseline: 1.3828x; 1.3828x over previous
import functools

import jax
import jax.numpy as jnp
from jax.experimental import pallas as pl
from jax.experimental.pallas import tpu as pltpu

_EPS = 1e-5
_BM = 320


def _norm_rows(z, f):
    zr = jnp.maximum(z, 0.0)
    mean = jnp.sum(zr, axis=1, keepdims=True) * (1.0 / f)
    diff = zr - mean
    var = jnp.sum(diff * diff, axis=1, keepdims=True) * (1.0 / max(f - 1, 1))
    return diff * pl.reciprocal(jnp.sqrt(var) + _EPS, approx=True)


def _body(x_ref, a_ref, w0_ref, b0_ref, w1_ref, b1_ref, w2_ref, b2_ref,
          out_ref, pen_ref,
          a32, xv, wv, bv, xwf, hf, outv, sem_a, sem_s, sem_o,
          *, n, f_in, fh, fo, nb):
    def slab(i):
        return pl.ds(i * _BM, _BM)

    a_cps = [pltpu.make_async_copy(a_ref.at[slab(i)], a32.at[slab(i)],
                                   sem_a.at[i]) for i in range(nb)]
    for cp in a_cps:
        cp.start()

    cp_x = pltpu.make_async_copy(x_ref, xv, sem_s.at[0])
    cp_w = pltpu.make_async_copy(w0_ref, wv.at[:f_in], sem_s.at[1])
    cp_b = pltpu.make_async_copy(b0_ref, bv, sem_s.at[2])
    cp_x.start()
    cp_w.start()
    cp_b.start()
    cp_x.wait()
    cp_w.wait()
    cp_b.wait()

    xwf[...] = jnp.dot(xv[...], wv[:f_in],
                       preferred_element_type=jnp.float32)

    for i in range(nb):
        a_cps[i].wait()
        z = jnp.dot(a32[slab(i)], xwf[...],
                    preferred_element_type=jnp.float32) + bv[...]
        hf[slab(i)] = _norm_rows(z, fh)

    cp_w = pltpu.make_async_copy(w1_ref, wv, sem_s.at[1])
    cp_b = pltpu.make_async_copy(b1_ref, bv, sem_s.at[2])
    cp_w.start()
    cp_b.start()
    cp_w.wait()
    cp_b.wait()
    xwf[...] = jnp.dot(hf[...], wv[...], preferred_element_type=jnp.float32)
    for i in range(nb):
        z = jnp.dot(a32[slab(i)], xwf[...],
                    preferred_element_type=jnp.float32) + bv[...]
        hf[slab(i)] = _norm_rows(z, fh)
    cp_pen = pltpu.make_async_copy(hf, pen_ref, sem_o.at[0])
    cp_pen.start()

    cp_w = pltpu.make_async_copy(w2_ref, wv, sem_s.at[1])
    cp_b = pltpu.make_async_copy(b2_ref, bv, sem_s.at[2])
    cp_w.start()
    cp_b.start()
    cp_w.wait()
    cp_b.wait()
    xwf[...] = jnp.dot(hf[...], wv[...], preferred_element_type=jnp.float32)
    for i in range(nb):
        z = jnp.dot(a32[slab(i)], xwf[...],
                    preferred_element_type=jnp.float32) + bv[...]
        outv[slab(i)] = z[:, :fo]
    cp_out = pltpu.make_async_copy(outv, out_ref, sem_o.at[1])
    cp_out.start()
    cp_pen.wait()
    cp_out.wait()


def kernel(x, a_hat, W0, b0, W1, b1, W2, b2):
    n, f_in = x.shape
    fh = W0.shape[1]
    fo = W2.shape[1]
    nb = n // _BM

    w2 = jnp.zeros((fh, fh), jnp.float32).at[:, :fo].set(W2)
    b2p = jnp.zeros((1, fh), jnp.float32).at[:, :fo].set(b2.reshape(1, -1))

    body = functools.partial(_body, n=n, f_in=f_in, fh=fh, fo=fo, nb=nb)
    flops = 3 * 2 * n * n * fh + 2 * n * (f_in + 2 * fh) * fh
    out, pen = pl.pallas_call(
        body,
        out_shape=(jax.ShapeDtypeStruct((n, fo), jnp.float32),
                   jax.ShapeDtypeStruct((n, fh), jnp.float32)),
        in_specs=[pl.BlockSpec(memory_space=pl.ANY)] * 8,
        out_specs=(pl.BlockSpec(memory_space=pl.ANY),
                   pl.BlockSpec(memory_space=pl.ANY)),
        scratch_shapes=[
            pltpu.VMEM((n, n), jnp.float32),
            pltpu.VMEM((n, f_in), jnp.float32),
            pltpu.VMEM((fh, fh), jnp.float32),
            pltpu.VMEM((1, fh), jnp.float32),
            pltpu.VMEM((n, fh), jnp.float32),
            pltpu.VMEM((n, fh), jnp.float32),
            pltpu.VMEM((n, fo), jnp.float32),
            pltpu.SemaphoreType.DMA((nb,)),
            pltpu.SemaphoreType.DMA((3,)),
            pltpu.SemaphoreType.DMA((2,)),
        ],
        compiler_params=pltpu.CompilerParams(
            vmem_limit_bytes=52 * 1024 * 1024,
        ),
        cost_estimate=pl.CostEstimate(
            flops=flops,
            transcendentals=2 * n,
            bytes_accessed=4 * (n * n + 4 * n * fh),
        ),
    )(x, a_hat, W0, b0.reshape(1, -1), W1, b1.reshape(1, -1), w2, b2p)
    return out, pen
